# baseline (device time: 51106 ns/iter reference)
import jax
import jax.numpy as jnp
from jax import lax
from jax.experimental import pallas as pl
from jax.experimental.pallas import tpu as pltpu

N_DEV = 4
B = 2
SQ = 512
SKV = 512
HQ = 8
DH = 64
D_MODEL = 768
D_QK = HQ * DH
BLK = 64
HALF = SKV // 2
NEG = -1e9
F32 = jnp.float32
BF16 = jnp.bfloat16
FP8 = jnp.float8_e4m3fn
LOG2E = 1.4426950408889634

S_K_R, S_V0_R, S_V1_R = 0, 1, 2
S_K_L, S_V1_L, S_V0_L = 3, 4, 5
S_K_H0, S_V_H0 = 6, 7
S_K_H1, S_V_H1 = 8, 9


def kernel(x, Wq, K_ext, V_ext, Wo):
    k2 = K_ext.reshape(B, SKV, D_QK)
    v2 = V_ext.reshape(B, SKV, D_QK)

    def body(x_ref, wq_ref, k_ref, v_ref, wo_ref, out_ref,
             kfull, vfull, send_sems, recv_sems):
        me = lax.axis_index("i")
        right = lax.rem(me + 1, N_DEV)
        left = lax.rem(me + 3, N_DEV)
        opp = lax.rem(me + 2, N_DEV)

        def rdma(src, dst, i, dev):
            return pltpu.make_async_remote_copy(
                src_ref=src, dst_ref=dst,
                send_sem=send_sems.at[i], recv_sem=recv_sems.at[i],
                device_id=(dev,), device_id_type=pl.DeviceIdType.MESH,
            )

        barrier_sem = pltpu.get_barrier_semaphore()
        for nbr in (left, right):
            pl.semaphore_signal(
                barrier_sem, inc=1,
                device_id=(nbr,), device_id_type=pl.DeviceIdType.MESH,
            )
        pl.semaphore_wait(barrier_sem, 2)

        kfull[me, 0] = k_ref[:, 0:HALF, :].astype(FP8)
        kfull[me, 1] = k_ref[:, HALF:SKV, :].astype(FP8)
        xfers = {
            S_K_R: rdma(kfull.at[me], kfull.at[me], S_K_R, right),
            S_K_L: rdma(kfull.at[me], kfull.at[me], S_K_L, left),
        }
        xfers[S_K_R].start()
        xfers[S_K_L].start()
        vfull[me, 0] = v_ref[:, 0:HALF, :].astype(BF16)
        xfers[S_V0_R] = rdma(vfull.at[me, 0], vfull.at[me, 0],
                             S_V0_R, right)
        xfers[S_V0_R].start()
        vfull[me, 1] = v_ref[:, HALF:SKV, :].astype(BF16)
        for i, hf, dev in ((S_V1_L, 1, left), (S_V1_R, 1, right),
                           (S_V0_L, 0, left)):
            xfers[i] = rdma(vfull.at[me, hf], vfull.at[me, hf], i, dev)
            xfers[i].start()

        wq16 = (wq_ref[...] * (0.125 * LOG2E)).astype(BF16)
        qp = [jnp.dot(x_ref[b].astype(BF16), wq16,
                      preferred_element_type=F32).astype(BF16)
              for b in range(B)]

        def chunk_mask(origin):
            r = lax.broadcasted_iota(jnp.int32, (SQ, SKV), 0)
            c = lax.broadcasted_iota(jnp.int32, (SQ, SKV), 1)
            qb = me * (SQ // BLK) + r // BLK
            kb = origin * (SKV // BLK) + c // BLK
            return (qb == kb) | (kb == 0) | (lax.rem(qb + kb, 3) == 0)

        masks = {name: chunk_mask(org)
                 for name, org in (("me", me), ("left", left),
                                   ("right", right), ("opp", opp))}

        state = {}
        ones_col = jnp.ones((HALF, 1), BF16)

        def process(mask_name, origin, half, local=False):
            lo = half * HALF
            mask = masks[mask_name][:, lo:lo + HALF]
            for b in range(B):
                if local:
                    kc = k_ref[b, lo:lo + HALF, :].astype(BF16)
                else:
                    kc = kfull[origin, half, b].astype(BF16)
                vc = vfull[origin, half, b]
                for h in range(HQ):
                    q = qp[b][:, h * DH:(h + 1) * DH]
                    k_o = kc[:, h * DH:(h + 1) * DH]
                    v_aug = jnp.concatenate(
                        [vc[:, h * DH:(h + 1) * DH], ones_col], axis=1)
                    s = lax.dot_general(
                        q, k_o, (((1,), (1,)), ((), ())),
                        preferred_element_type=F32)
                    w = jnp.exp2(jnp.where(mask, s, NEG))
                    aug = jnp.dot(w.astype(BF16), v_aug,
                                  preferred_element_type=F32)
                    if (b, h) not in state:
                        state[(b, h)] = aug
                    else:
                        state[(b, h)] = state[(b, h)] + aug

        process("me", me, 0, local=True)
        process("me", me, 1, local=True)

        xfers[S_K_R].wait_recv()
        h2 = {S_K_H0: rdma(kfull.at[left, 0], kfull.at[left, 0],
                           S_K_H0, right)}
        h2[S_K_H0].start()
        xfers[S_K_L].wait_recv()
        h2[S_K_H1] = rdma(kfull.at[right, 1], kfull.at[right, 1],
                          S_K_H1, left)
        h2[S_K_H1].start()
        xfers[S_V0_R].wait_recv()
        h2[S_V_H0] = rdma(vfull.at[left, 0], vfull.at[left, 0],
                          S_V_H0, right)
        h2[S_V_H0].start()
        xfers[S_V1_L].wait_recv()
        h2[S_V_H1] = rdma(vfull.at[right, 1], vfull.at[right, 1],
                          S_V_H1, left)
        h2[S_V_H1].start()

        process("left", left, 0)
        process("right", right, 1)
        xfers[S_V1_R].wait_recv()
        process("left", left, 1)
        xfers[S_V0_L].wait_recv()
        process("right", right, 0)

        h2[S_K_H0].wait_recv()
        h2[S_V_H0].wait_recv()
        process("opp", opp, 0)
        h2[S_K_H1].wait_recv()
        h2[S_V_H1].wait_recv()
        process("opp", opp, 1)

        wo_b = wo_ref[...].astype(BF16)
        for b in range(B):
            ctx = jnp.concatenate(
                [state[(b, h)][:, :DH] / state[(b, h)][:, DH:DH + 1]
                 for h in range(HQ)],
                axis=1)
            out_ref[b] = jnp.dot(ctx.astype(BF16), wo_b,
                                 preferred_element_type=F32)

        for r in list(xfers.values()) + list(h2.values()):
            r.wait_send()

    return pl.pallas_call(
        body,
        out_shape=jax.ShapeDtypeStruct((B, SQ, D_MODEL), jnp.float32),
        in_specs=[pl.BlockSpec(memory_space=pltpu.VMEM)] * 5,
        out_specs=pl.BlockSpec(memory_space=pltpu.VMEM),
        scratch_shapes=[
            pltpu.VMEM((N_DEV, 2, B, HALF, D_QK), FP8),
            pltpu.VMEM((N_DEV, 2, B, HALF, D_QK), BF16),
            pltpu.SemaphoreType.DMA((10,)),
            pltpu.SemaphoreType.DMA((10,)),
        ],
        compiler_params=pltpu.CompilerParams(
            vmem_limit_bytes=100 * 1024 * 1024,
            collective_id=0,
        ),
    )(x, Wq, k2, v2, Wo)


# device time: 51084 ns/iter; 1.0004x vs baseline; 1.0004x over previous
import jax
import jax.numpy as jnp
from jax import lax
from jax.experimental import pallas as pl
from jax.experimental.pallas import tpu as pltpu

N_DEV = 4
B = 2
SQ = 512
SKV = 512
HQ = 8
DH = 64
D_MODEL = 768
D_QK = HQ * DH
BLK = 64
HALF = SKV // 2
NEG = -1e9
F32 = jnp.float32
BF16 = jnp.bfloat16
FP8 = jnp.float8_e4m3fn
LOG2E = 1.4426950408889634

S_K_R, S_V0_R, S_V1_R = 0, 1, 2
S_K_L, S_V1_L, S_V0_L = 3, 4, 5
S_K_H0, S_V_H0 = 6, 7
S_K_H1, S_V_H1 = 8, 9


def kernel(x, Wq, K_ext, V_ext, Wo):
    k2 = K_ext.reshape(B, SKV, D_QK)
    v2 = V_ext.reshape(B, SKV, D_QK)

    def body(x_ref, wq_ref, k_ref, v_ref, wo_ref, out_ref,
             kfull, vfull, send_sems, recv_sems):
        me = lax.axis_index("i")
        right = lax.rem(me + 1, N_DEV)
        left = lax.rem(me + 3, N_DEV)
        opp = lax.rem(me + 2, N_DEV)

        def rdma(src, dst, i, dev):
            return pltpu.make_async_remote_copy(
                src_ref=src, dst_ref=dst,
                send_sem=send_sems.at[i], recv_sem=recv_sems.at[i],
                device_id=(dev,), device_id_type=pl.DeviceIdType.MESH,
            )

        barrier_sem = pltpu.get_barrier_semaphore()
        for nbr in (left, right):
            pl.semaphore_signal(
                barrier_sem, inc=1,
                device_id=(nbr,), device_id_type=pl.DeviceIdType.MESH,
            )
        pl.semaphore_wait(barrier_sem, 2)

        kfull[me, 0] = k_ref[:, 0:HALF, :].astype(FP8)
        kfull[me, 1] = k_ref[:, HALF:SKV, :].astype(FP8)
        xfers = {
            S_K_R: rdma(kfull.at[me], kfull.at[me], S_K_R, right),
            S_K_L: rdma(kfull.at[me], kfull.at[me], S_K_L, left),
        }
        xfers[S_K_R].start()
        xfers[S_K_L].start()
        vfull[me, 0] = v_ref[:, 0:HALF, :].astype(BF16)
        xfers[S_V0_R] = rdma(vfull.at[me, 0], vfull.at[me, 0],
                             S_V0_R, right)
        xfers[S_V0_R].start()
        vfull[me, 1] = v_ref[:, HALF:SKV, :].astype(BF16)
        xfers[S_V1_L] = rdma(vfull.at[me, 1], vfull.at[me, 1],
                             S_V1_L, left)
        xfers[S_V1_L].start()

        wq16 = (wq_ref[...] * (0.125 * LOG2E)).astype(BF16)
        qp = [jnp.dot(x_ref[b].astype(BF16), wq16,
                      preferred_element_type=F32).astype(BF16)
              for b in range(B)]

        def chunk_mask(origin):
            r = lax.broadcasted_iota(jnp.int32, (SQ, SKV), 0)
            c = lax.broadcasted_iota(jnp.int32, (SQ, SKV), 1)
            qb = me * (SQ // BLK) + r // BLK
            kb = origin * (SKV // BLK) + c // BLK
            return (qb == kb) | (kb == 0) | (lax.rem(qb + kb, 3) == 0)

        masks = {name: chunk_mask(org)
                 for name, org in (("me", me), ("left", left),
                                   ("right", right), ("opp", opp))}

        state = {}
        ones_col = jnp.ones((HALF, 1), BF16)

        def process(mask_name, origin, half, local=False):
            lo = half * HALF
            mask = masks[mask_name][:, lo:lo + HALF]
            for b in range(B):
                if local:
                    kc = k_ref[b, lo:lo + HALF, :].astype(BF16)
                else:
                    kc = kfull[origin, half, b].astype(BF16)
                vc = vfull[origin, half, b]
                for h in range(HQ):
                    q = qp[b][:, h * DH:(h + 1) * DH]
                    k_o = kc[:, h * DH:(h + 1) * DH]
                    v_aug = jnp.concatenate(
                        [vc[:, h * DH:(h + 1) * DH], ones_col], axis=1)
                    s = lax.dot_general(
                        q, k_o, (((1,), (1,)), ((), ())),
                        preferred_element_type=F32)
                    w = jnp.exp2(jnp.where(mask, s, NEG))
                    aug = jnp.dot(w.astype(BF16), v_aug,
                                  preferred_element_type=F32)
                    if (b, h) not in state:
                        state[(b, h)] = aug
                    else:
                        state[(b, h)] = state[(b, h)] + aug

        process("me", me, 0, local=True)
        process("me", me, 1, local=True)

        xfers[S_K_R].wait_recv()
        h2 = {S_K_H0: rdma(kfull.at[left, 0], kfull.at[left, 0],
                           S_K_H0, right)}
        h2[S_K_H0].start()
        xfers[S_K_L].wait_recv()
        h2[S_K_H1] = rdma(kfull.at[right, 1], kfull.at[right, 1],
                          S_K_H1, left)
        h2[S_K_H1].start()
        xfers[S_V0_R].wait_recv()
        h2[S_V_H0] = rdma(vfull.at[left, 0], vfull.at[left, 0],
                          S_V_H0, right)
        h2[S_V_H0].start()
        xfers[S_V1_R] = rdma(vfull.at[me, 1], vfull.at[me, 1],
                             S_V1_R, right)
        xfers[S_V1_R].start()
        xfers[S_V1_L].wait_recv()
        h2[S_V_H1] = rdma(vfull.at[right, 1], vfull.at[right, 1],
                          S_V_H1, left)
        h2[S_V_H1].start()
        xfers[S_V0_L] = rdma(vfull.at[me, 0], vfull.at[me, 0],
                             S_V0_L, left)
        xfers[S_V0_L].start()

        process("left", left, 0)
        process("right", right, 1)

        h2[S_K_H0].wait_recv()
        h2[S_V_H0].wait_recv()
        process("opp", opp, 0)
        h2[S_K_H1].wait_recv()
        h2[S_V_H1].wait_recv()
        process("opp", opp, 1)

        xfers[S_V1_R].wait_recv()
        process("left", left, 1)
        xfers[S_V0_L].wait_recv()
        process("right", right, 0)

        wo_b = wo_ref[...].astype(BF16)
        for b in range(B):
            ctx = jnp.concatenate(
                [state[(b, h)][:, :DH] / state[(b, h)][:, DH:DH + 1]
                 for h in range(HQ)],
                axis=1)
            out_ref[b] = jnp.dot(ctx.astype(BF16), wo_b,
                                 preferred_element_type=F32)

        for r in list(xfers.values()) + list(h2.values()):
            r.wait_send()

    return pl.pallas_call(
        body,
        out_shape=jax.ShapeDtypeStruct((B, SQ, D_MODEL), jnp.float32),
        in_specs=[pl.BlockSpec(memory_space=pltpu.VMEM)] * 5,
        out_specs=pl.BlockSpec(memory_space=pltpu.VMEM),
        scratch_shapes=[
            pltpu.VMEM((N_DEV, 2, B, HALF, D_QK), FP8),
            pltpu.VMEM((N_DEV, 2, B, HALF, D_QK), BF16),
            pltpu.SemaphoreType.DMA((10,)),
            pltpu.SemaphoreType.DMA((10,)),
        ],
        compiler_params=pltpu.CompilerParams(
            vmem_limit_bytes=100 * 1024 * 1024,
            collective_id=0,
        ),
    )(x, Wq, k2, v2, Wo)


# device time: 50782 ns/iter; 1.0064x vs baseline; 1.0059x over previous
import jax
import jax.numpy as jnp
from jax import lax
from jax.experimental import pallas as pl
from jax.experimental.pallas import tpu as pltpu

N_DEV = 4
B = 2
SQ = 512
SKV = 512
HQ = 8
DH = 64
D_MODEL = 768
D_QK = HQ * DH
BLK = 64
HALF = SKV // 2
NEG = -1e9
F32 = jnp.float32
BF16 = jnp.bfloat16
FP8 = jnp.float8_e4m3fn
LOG2E = 1.4426950408889634

S_K_R, S_V0_R, S_V1_R = 0, 1, 2
S_K_L, S_V1_L, S_V0_L = 3, 4, 5
S_K_H0, S_V_H0 = 6, 7
S_K_H1, S_V_H1 = 8, 9


def kernel(x, Wq, K_ext, V_ext, Wo):
    k2 = K_ext.reshape(B, SKV, D_QK)
    v2 = V_ext.reshape(B, SKV, D_QK)

    def body(x_ref, wq_ref, k_ref, v_ref, wo_ref, out_ref,
             kfull, vfull, send_sems, recv_sems):
        me = lax.axis_index("i")
        right = lax.rem(me + 1, N_DEV)
        left = lax.rem(me + 3, N_DEV)
        opp = lax.rem(me + 2, N_DEV)

        def rdma(src, dst, i, dev):
            return pltpu.make_async_remote_copy(
                src_ref=src, dst_ref=dst,
                send_sem=send_sems.at[i], recv_sem=recv_sems.at[i],
                device_id=(dev,), device_id_type=pl.DeviceIdType.MESH,
            )

        barrier_sem = pltpu.get_barrier_semaphore()
        for nbr in (left, right):
            pl.semaphore_signal(
                barrier_sem, inc=1,
                device_id=(nbr,), device_id_type=pl.DeviceIdType.MESH,
            )
        pl.semaphore_wait(barrier_sem, 2)

        def vh(buf, slot, half):
            return buf.at[slot, :, pl.ds(half * HALF, HALF), :]

        kfull[me] = k_ref[...].astype(FP8)
        xfers = {
            S_K_R: rdma(kfull.at[me], kfull.at[me], S_K_R, right),
            S_K_L: rdma(kfull.at[me], kfull.at[me], S_K_L, left),
        }
        xfers[S_K_R].start()
        xfers[S_K_L].start()
        vfull[me] = v_ref[...].astype(BF16)
        for i, slot_half, dev in (
            (S_V0_R, 0, right), (S_V1_L, 1, left),
            (S_V1_R, 1, right), (S_V0_L, 0, left),
        ):
            xfers[i] = rdma(vh(vfull, me, slot_half),
                            vh(vfull, me, slot_half), i, dev)
            xfers[i].start()

        wq16 = (wq_ref[...] * (0.125 * LOG2E)).astype(BF16)
        qp = [jnp.dot(x_ref[b].astype(BF16), wq16,
                      preferred_element_type=F32).astype(BF16)
              for b in range(B)]

        def chunk_mask(origin):
            r = lax.broadcasted_iota(jnp.int32, (SQ, SKV), 0)
            c = lax.broadcasted_iota(jnp.int32, (SQ, SKV), 1)
            qb = me * (SQ // BLK) + r // BLK
            kb = origin * (SKV // BLK) + c // BLK
            return (qb == kb) | (kb == 0) | (lax.rem(qb + kb, 3) == 0)

        masks = {name: chunk_mask(org)
                 for name, org in (("me", me), ("left", left),
                                   ("right", right), ("opp", opp))}

        state = {}
        ones_col = jnp.ones((SKV, 1), BF16)

        def process(mask_name, kc_fn, vc_fn, half=None):
            lo, sz = (0, SKV) if half is None else (half * HALF, HALF)
            mask = masks[mask_name][:, lo:lo + sz]
            for b in range(B):
                kc = kc_fn(b)[lo:lo + sz, :]
                vc = vc_fn(b)[lo:lo + sz, :]
                for h in range(HQ):
                    q = qp[b][:, h * DH:(h + 1) * DH]
                    k_o = kc[:, h * DH:(h + 1) * DH]
                    v_aug = jnp.concatenate(
                        [vc[:, h * DH:(h + 1) * DH], ones_col[:sz]],
                        axis=1)
                    s = lax.dot_general(
                        q, k_o, (((1,), (1,)), ((), ())),
                        preferred_element_type=F32)
                    w = jnp.exp2(jnp.where(mask, s, NEG))
                    aug = jnp.dot(w.astype(BF16), v_aug,
                                  preferred_element_type=F32)
                    if (b, h) not in state:
                        state[(b, h)] = aug
                    else:
                        state[(b, h)] = state[(b, h)] + aug

        def remote(origin):
            return (lambda b: kfull[origin, b].astype(BF16),
                    lambda b: vfull[origin, b])

        process("me",
                lambda b: k_ref[b].astype(BF16),
                lambda b: vfull[me, b])

        xfers[S_K_R].wait_recv()
        xfers[S_V0_R].wait_recv()
        h2r = [rdma(vh(kfull, left, 0), vh(kfull, left, 0),
                    S_K_H0, right),
               rdma(vh(vfull, left, 0), vh(vfull, left, 0),
                    S_V_H0, right)]
        for r in h2r:
            r.start()
        xfers[S_K_L].wait_recv()
        xfers[S_V1_L].wait_recv()
        h2l = [rdma(vh(kfull, right, 1), vh(kfull, right, 1),
                    S_K_H1, left),
               rdma(vh(vfull, right, 1), vh(vfull, right, 1),
                    S_V_H1, left)]
        for r in h2l:
            r.start()

        process("left", *remote(left), half=0)
        xfers[S_V1_R].wait_recv()
        process("left", *remote(left), half=1)
        process("right", *remote(right), half=1)
        xfers[S_V0_L].wait_recv()
        process("right", *remote(right), half=0)

        for r in h2r + h2l:
            r.wait_recv()
        process("opp", *remote(opp))

        wo_b = wo_ref[...].astype(BF16)
        for b in range(B):
            ctx = jnp.concatenate(
                [state[(b, h)][:, :DH] / state[(b, h)][:, DH:DH + 1]
                 for h in range(HQ)],
                axis=1)
            out_ref[b] = jnp.dot(ctx.astype(BF16), wo_b,
                                 preferred_element_type=F32)

        for r in list(xfers.values()) + h2r + h2l:
            r.wait_send()

    return pl.pallas_call(
        body,
        out_shape=jax.ShapeDtypeStruct((B, SQ, D_MODEL), jnp.float32),
        in_specs=[pl.BlockSpec(memory_space=pltpu.VMEM)] * 5,
        out_specs=pl.BlockSpec(memory_space=pltpu.VMEM),
        scratch_shapes=[
            pltpu.VMEM((N_DEV, B, SKV, D_QK), FP8),
            pltpu.VMEM((N_DEV, B, SKV, D_QK), BF16),
            pltpu.SemaphoreType.DMA((10,)),
            pltpu.SemaphoreType.DMA((10,)),
        ],
        compiler_params=pltpu.CompilerParams(
            vmem_limit_bytes=100 * 1024 * 1024,
            collective_id=0,
        ),
    )(x, Wq, k2, v2, Wo)
